# baseline (device time: 62191 ns/iter reference)
import jax
import jax.numpy as jnp
from jax import lax
from jax.experimental import pallas as pl
from jax.experimental.pallas import tpu as pltpu

N_DEV = 4
B, SQ, SKV = 2, 512, 512
HQ, DH = 8, 64
D_MODEL = 768
D_HEADS = HQ * DH
BLK = 64
STRIDE = 4


def kernel(x, Wq, K_ext, V_ext, Wo):
    my = lax.axis_index("i")
    K = lax.dynamic_slice_in_dim(K_ext, my * HQ, HQ, axis=2)
    V = lax.dynamic_slice_in_dim(V_ext, my * HQ, HQ, axis=2)

    def body(x_ref, wq_ref, k_ref, v_ref, wo_ref, out_ref,
             comm_ref, send_sems, recv_sems):
        my_pos = lax.axis_index("i")

        xb = x_ref[...].reshape(B * SQ, D_MODEL).astype(jnp.bfloat16)
        wq = wq_ref[...].astype(jnp.bfloat16)
        q_all = jnp.dot(xb, wq, preferred_element_type=jnp.float32)

        qb = lax.broadcasted_iota(jnp.int32, (SQ, SKV), 0) // BLK
        kb = lax.broadcasted_iota(jnp.int32, (SQ, SKV), 1) // BLK
        mask = (qb % STRIDE) == (kb % STRIDE)

        ctx_rows = []
        for b in range(B):
            k_b = k_ref[b].astype(jnp.bfloat16)
            v_b = v_ref[b].astype(jnp.bfloat16)
            heads = []
            for h in range(HQ):
                qh = q_all[b * SQ:(b + 1) * SQ, h * DH:(h + 1) * DH]
                qh = qh.astype(jnp.bfloat16)
                kh = k_b[:, h, :]
                vh = v_b[:, h, :]
                s = lax.dot_general(
                    qh, kh, (((1,), (1,)), ((), ())),
                    preferred_element_type=jnp.float32,
                ) * 0.125
                s = jnp.where(mask, s, -1e9)
                m = jnp.max(s, axis=1, keepdims=True)
                w = jnp.exp(s - m)
                w = w / jnp.sum(w, axis=1, keepdims=True)
                ctx = jnp.dot(w.astype(jnp.bfloat16), vh,
                              preferred_element_type=jnp.float32)
                heads.append(ctx.astype(jnp.bfloat16))
            ctx_rows.append(jnp.concatenate(heads, axis=1))
        ctx_all = jnp.concatenate(ctx_rows, axis=0)

        wo = wo_ref[...].astype(jnp.bfloat16)
        partial = jnp.dot(ctx_all, wo, preferred_element_type=jnp.float32)

        out_ref[...] = partial.reshape(B, SQ, D_MODEL)
        comm_ref[0] = partial.astype(jnp.bfloat16)

        barrier = pltpu.get_barrier_semaphore()
        for d in range(1, N_DEV):
            pl.semaphore_signal(
                barrier, inc=1,
                device_id=((my_pos + d) % N_DEV,),
                device_id_type=pl.DeviceIdType.MESH,
            )
        pl.semaphore_wait(barrier, N_DEV - 1)

        sends = []
        for d in range(1, N_DEV):
            rdma = pltpu.make_async_remote_copy(
                src_ref=comm_ref.at[0],
                dst_ref=comm_ref.at[N_DEV - d],
                send_sem=send_sems.at[d - 1],
                recv_sem=recv_sems.at[N_DEV - d],
                device_id=((my_pos + d) % N_DEV,),
                device_id_type=pl.DeviceIdType.MESH,
            )
            rdma.start()
            sends.append(rdma)

        for j in range(1, N_DEV):
            recv = pltpu.make_async_remote_copy(
                src_ref=comm_ref.at[0],
                dst_ref=comm_ref.at[j],
                send_sem=send_sems.at[0],
                recv_sem=recv_sems.at[j],
                device_id=(my_pos,),
                device_id_type=pl.DeviceIdType.MESH,
            )
            recv.wait_recv()
            out_ref[...] += comm_ref[j].astype(jnp.float32).reshape(
                B, SQ, D_MODEL)

        for rdma in sends:
            rdma.wait_send()

    return pl.pallas_call(
        body,
        out_shape=jax.ShapeDtypeStruct((B, SQ, D_MODEL), jnp.float32),
        in_specs=[pl.BlockSpec(memory_space=pltpu.VMEM)] * 5,
        out_specs=pl.BlockSpec(memory_space=pltpu.VMEM),
        scratch_shapes=[
            pltpu.VMEM((N_DEV, B * SQ, D_MODEL), jnp.bfloat16),
            pltpu.SemaphoreType.DMA((N_DEV - 1,)),
            pltpu.SemaphoreType.DMA((N_DEV,)),
        ],
        compiler_params=pltpu.CompilerParams(collective_id=0),
    )(x, Wq, K, V, Wo)


# device time: 53433 ns/iter; 1.1639x vs baseline; 1.1639x over previous
import jax
import jax.numpy as jnp
from jax import lax
from jax.experimental import pallas as pl
from jax.experimental.pallas import tpu as pltpu

N_DEV = 4
B, SQ, SKV = 2, 512, 512
HQ, DH = 8, 64
D_MODEL = 768
BLK = 64
STRIDE = 4
ROWS = B * SQ
CH = ROWS // N_DEV


def kernel(x, Wq, K_ext, V_ext, Wo):
    my = lax.axis_index("i")
    K = lax.dynamic_slice_in_dim(K_ext, my * HQ, HQ, axis=2)
    V = lax.dynamic_slice_in_dim(V_ext, my * HQ, HQ, axis=2)

    def body(x_ref, wq_ref, k_ref, v_ref, wo_ref, out_ref,
             send_buf, rs_ref, ag_ref, rs_send, rs_recv, ag_send, ag_recv):
        my_pos = lax.axis_index("i")

        xb = x_ref[...].reshape(ROWS, D_MODEL).astype(jnp.bfloat16)
        wq = wq_ref[...].astype(jnp.bfloat16)
        q_all = jnp.dot(xb, wq, preferred_element_type=jnp.float32)
        wo = wo_ref[...].astype(jnp.bfloat16)

        def grp(a, c):
            return jnp.concatenate(
                [a[BLK * c:BLK * (c + 1)],
                 a[4 * BLK + BLK * c:4 * BLK + BLK * (c + 1)]], axis=0)

        def attn(b):
            k_b = k_ref[b].astype(jnp.bfloat16)
            v_b = v_ref[b].astype(jnp.bfloat16)
            heads = []
            for h in range(HQ):
                qh = q_all[b * SQ:(b + 1) * SQ, h * DH:(h + 1) * DH]
                qh = qh.astype(jnp.bfloat16)
                kh = k_b[:, h, :]
                vh = v_b[:, h, :]
                parts = []
                for c in range(STRIDE):
                    qg = grp(qh, c)
                    kg = grp(kh, c)
                    vg = grp(vh, c)
                    s = lax.dot_general(
                        qg, kg, (((1,), (1,)), ((), ())),
                        preferred_element_type=jnp.float32) * 0.125
                    m = jnp.max(s, axis=1, keepdims=True)
                    w = jnp.exp(s - m)
                    w = w / jnp.sum(w, axis=1, keepdims=True)
                    ctx = jnp.dot(w.astype(jnp.bfloat16), vg,
                                  preferred_element_type=jnp.float32)
                    parts.append(ctx.astype(jnp.bfloat16))
                ctx_h = jnp.concatenate(
                    [parts[c][BLK * half:BLK * (half + 1)]
                     for half in range(2) for c in range(STRIDE)], axis=0)
                heads.append(ctx_h)
            return jnp.concatenate(heads, axis=1)

        ctx0 = attn(0)

        barrier = pltpu.get_barrier_semaphore()
        for d in range(1, N_DEV):
            pl.semaphore_signal(
                barrier, inc=1,
                device_id=((my_pos + d) % N_DEV,),
                device_id_type=pl.DeviceIdType.MESH,
            )
        pl.semaphore_wait(barrier, N_DEV - 1)

        pcs = []
        ctxs = {0: ctx0}

        def chunk(q):
            b, half = divmod(q, 2)
            pc = jnp.dot(ctxs[b][half * CH:(half + 1) * CH, :], wo,
                         preferred_element_type=jnp.float32)
            pcs.append(pc)
            send_buf[q] = pc.astype(jnp.bfloat16)
            o = (my_pos - q) % N_DEV

            @pl.when(o != 0)
            def _():
                rdma = pltpu.make_async_remote_copy(
                    src_ref=send_buf.at[q],
                    dst_ref=rs_ref.at[o],
                    send_sem=rs_send.at[q],
                    recv_sem=rs_recv.at[o],
                    device_id=(q,),
                    device_id_type=pl.DeviceIdType.MESH,
                )
                rdma.start()

        chunk(0)
        chunk(1)
        ctxs[1] = attn(1)
        chunk(2)
        chunk(3)

        own = jnp.where(
            my_pos < 2,
            jnp.where(my_pos == 0, pcs[0], pcs[1]),
            jnp.where(my_pos == 2, pcs[2], pcs[3]))
        red = own
        for o in range(1, N_DEV):
            recv = pltpu.make_async_remote_copy(
                src_ref=send_buf.at[0], dst_ref=rs_ref.at[o],
                send_sem=rs_send.at[0], recv_sem=rs_recv.at[o],
                device_id=(my_pos,), device_id_type=pl.DeviceIdType.MESH,
            )
            recv.wait_recv()
            red = red + rs_ref[o].astype(jnp.float32)

        out_ref[pl.ds(my_pos * CH, CH), :] = red
        ag_ref[0] = red.astype(jnp.bfloat16)

        ag_sends = []
        for d in range(1, N_DEV):
            rdma = pltpu.make_async_remote_copy(
                src_ref=ag_ref.at[0],
                dst_ref=ag_ref.at[N_DEV - d],
                send_sem=ag_send.at[d - 1],
                recv_sem=ag_recv.at[N_DEV - d],
                device_id=((my_pos + d) % N_DEV,),
                device_id_type=pl.DeviceIdType.MESH,
            )
            rdma.start()
            ag_sends.append(rdma)

        for o in range(1, N_DEV):
            recv = pltpu.make_async_remote_copy(
                src_ref=ag_ref.at[0], dst_ref=ag_ref.at[o],
                send_sem=ag_send.at[0], recv_sem=ag_recv.at[o],
                device_id=(my_pos,), device_id_type=pl.DeviceIdType.MESH,
            )
            recv.wait_recv()
            p = (my_pos + o) % N_DEV
            out_ref[pl.ds(p * CH, CH), :] = ag_ref[o].astype(jnp.float32)

        for q in range(N_DEV):
            o = (my_pos - q) % N_DEV

            @pl.when(o != 0)
            def _():
                rdma = pltpu.make_async_remote_copy(
                    src_ref=send_buf.at[q], dst_ref=rs_ref.at[1],
                    send_sem=rs_send.at[q], recv_sem=rs_recv.at[1],
                    device_id=(q,), device_id_type=pl.DeviceIdType.MESH,
                )
                rdma.wait_send()
        for rdma in ag_sends:
            rdma.wait_send()

    out = pl.pallas_call(
        body,
        out_shape=jax.ShapeDtypeStruct((ROWS, D_MODEL), jnp.float32),
        in_specs=[pl.BlockSpec(memory_space=pltpu.VMEM)] * 5,
        out_specs=pl.BlockSpec(memory_space=pltpu.VMEM),
        scratch_shapes=[
            pltpu.VMEM((N_DEV, CH, D_MODEL), jnp.bfloat16),
            pltpu.VMEM((N_DEV, CH, D_MODEL), jnp.bfloat16),
            pltpu.VMEM((N_DEV, CH, D_MODEL), jnp.bfloat16),
            pltpu.SemaphoreType.DMA((N_DEV,)),
            pltpu.SemaphoreType.DMA((N_DEV,)),
            pltpu.SemaphoreType.DMA((N_DEV - 1,)),
            pltpu.SemaphoreType.DMA((N_DEV,)),
        ],
        compiler_params=pltpu.CompilerParams(collective_id=0),
    )(x, Wq, K, V, Wo)
    return out.reshape(B, SQ, D_MODEL)


# device time: 47319 ns/iter; 1.3143x vs baseline; 1.1292x over previous
import jax
import jax.numpy as jnp
from jax import lax
from jax.experimental import pallas as pl
from jax.experimental.pallas import tpu as pltpu

N_DEV = 4
B, SQ, SKV = 2, 512, 512
HQ, DH = 8, 64
D_MODEL = 768
BLK = 64
STRIDE = 4
ROWS = B * SQ
CH = ROWS // N_DEV


def kernel(x, Wq, K_ext, V_ext, Wo):
    my = lax.axis_index("i")
    K = lax.dynamic_slice_in_dim(K_ext, my * HQ, HQ, axis=2)
    V = lax.dynamic_slice_in_dim(V_ext, my * HQ, HQ, axis=2)
    K = K.astype(jnp.bfloat16)
    V = V.astype(jnp.bfloat16)
    xb = x.reshape(ROWS, D_MODEL).astype(jnp.bfloat16)
    wq = (Wq * 0.125).astype(jnp.bfloat16)
    wo = Wo.astype(jnp.bfloat16)

    def body(x_ref, wq_ref, k_ref, v_ref, wo_ref, out_ref,
             q_ref, send_buf, rs_ref, ag_ref,
             rs_send, rs_recv, ag_send, ag_recv):
        my_pos = lax.axis_index("i")

        q_all = jnp.dot(x_ref[...], wq_ref[...],
                        preferred_element_type=jnp.float32)
        q_ref[...] = q_all.astype(jnp.bfloat16)

        qbi = lax.broadcasted_iota(jnp.int32, (CH, SKV), 0) // BLK
        kbi = lax.broadcasted_iota(jnp.int32, (CH, SKV), 1) // BLK
        mask = (qbi % STRIDE) == (kbi % STRIDE)

        barrier = pltpu.get_barrier_semaphore()
        for d in range(1, N_DEV):
            pl.semaphore_signal(
                barrier, inc=1,
                device_id=((my_pos + d) % N_DEV,),
                device_id_type=pl.DeviceIdType.MESH,
            )
        pl.semaphore_wait(barrier, N_DEV - 1)

        own_pc = None
        for i in range(N_DEV):
            q_idx = (my_pos + 1 + i) % N_DEV
            b = q_idx // 2
            qm = q_ref[pl.ds(q_idx * CH, CH), :]
            k_b = k_ref[b]
            v_b = v_ref[b]
            heads = []
            for h in range(HQ):
                qh = qm[:, h * DH:(h + 1) * DH]
                kh = k_b[:, h, :]
                vh = v_b[:, h, :]
                s = lax.dot_general(qh, kh, (((1,), (1,)), ((), ())),
                                    preferred_element_type=jnp.float32)
                w = jnp.exp(jnp.where(mask, s, -40.0))
                w = w / jnp.sum(w, axis=1, keepdims=True)
                ctx = jnp.dot(w.astype(jnp.bfloat16), vh,
                              preferred_element_type=jnp.float32)
                heads.append(ctx.astype(jnp.bfloat16))
            ctx_m = jnp.concatenate(heads, axis=1)
            pc = jnp.dot(ctx_m, wo_ref[...],
                         preferred_element_type=jnp.float32)
            if i < N_DEV - 1:
                send_buf[N_DEV - 1 - i] = pc.astype(jnp.bfloat16)
                rdma = pltpu.make_async_remote_copy(
                    src_ref=send_buf.at[N_DEV - 1 - i],
                    dst_ref=rs_ref.at[N_DEV - 1 - i],
                    send_sem=rs_send.at[N_DEV - 1 - i],
                    recv_sem=rs_recv.at[N_DEV - 1 - i],
                    device_id=(q_idx,),
                    device_id_type=pl.DeviceIdType.MESH,
                )
                rdma.start()
            else:
                own_pc = pc

        red = own_pc
        for o in range(1, N_DEV):
            recv = pltpu.make_async_remote_copy(
                src_ref=send_buf.at[o], dst_ref=rs_ref.at[o],
                send_sem=rs_send.at[o], recv_sem=rs_recv.at[o],
                device_id=(my_pos,), device_id_type=pl.DeviceIdType.MESH,
            )
            recv.wait_recv()
            red = red + rs_ref[o].astype(jnp.float32)

        out_ref[pl.ds(my_pos * CH, CH), :] = red
        ag_ref[0] = red.astype(jnp.bfloat16)

        ag_sends = []
        for d in range(1, N_DEV):
            rdma = pltpu.make_async_remote_copy(
                src_ref=ag_ref.at[0],
                dst_ref=ag_ref.at[N_DEV - d],
                send_sem=ag_send.at[d - 1],
                recv_sem=ag_recv.at[N_DEV - d],
                device_id=((my_pos + d) % N_DEV,),
                device_id_type=pl.DeviceIdType.MESH,
            )
            rdma.start()
            ag_sends.append(rdma)

        for o in range(1, N_DEV):
            recv = pltpu.make_async_remote_copy(
                src_ref=ag_ref.at[0], dst_ref=ag_ref.at[o],
                send_sem=ag_send.at[0], recv_sem=ag_recv.at[o],
                device_id=(my_pos,), device_id_type=pl.DeviceIdType.MESH,
            )
            recv.wait_recv()
            p = (my_pos + o) % N_DEV
            out_ref[pl.ds(p * CH, CH), :] = ag_ref[o].astype(jnp.float32)

        for o in range(1, N_DEV):
            rdma = pltpu.make_async_remote_copy(
                src_ref=send_buf.at[o], dst_ref=rs_ref.at[o],
                send_sem=rs_send.at[o], recv_sem=rs_recv.at[o],
                device_id=(my_pos,), device_id_type=pl.DeviceIdType.MESH,
            )
            rdma.wait_send()
        for rdma in ag_sends:
            rdma.wait_send()

    out = pl.pallas_call(
        body,
        out_shape=jax.ShapeDtypeStruct((ROWS, D_MODEL), jnp.float32),
        in_specs=[pl.BlockSpec(memory_space=pltpu.VMEM)] * 5,
        out_specs=pl.BlockSpec(memory_space=pltpu.VMEM),
        scratch_shapes=[
            pltpu.VMEM((ROWS, HQ * DH), jnp.bfloat16),
            pltpu.VMEM((N_DEV, CH, D_MODEL), jnp.bfloat16),
            pltpu.VMEM((N_DEV, CH, D_MODEL), jnp.bfloat16),
            pltpu.VMEM((N_DEV, CH, D_MODEL), jnp.bfloat16),
            pltpu.SemaphoreType.DMA((N_DEV,)),
            pltpu.SemaphoreType.DMA((N_DEV,)),
            pltpu.SemaphoreType.DMA((N_DEV - 1,)),
            pltpu.SemaphoreType.DMA((N_DEV,)),
        ],
        compiler_params=pltpu.CompilerParams(collective_id=0),
    )(xb, wq, K, V, wo)
    return out.reshape(B, SQ, D_MODEL)
